# baseline (speedup 1.0000x reference)
import functools

import jax
import jax.numpy as jnp
from jax import lax
from jax.experimental import pallas as pl
from jax.experimental.pallas import tpu as pltpu
from jax.experimental.pallas import tpu_sc as plsc

_B, _S, _D = 4, 8192, 1024
_NC, _NS, _L = 2, 16, 16
_NW = _NC * _NS
_ROWS_W = _S // _NW
_R = 32
_CHUNK = _R * _D
_NCHUNK = _ROWS_W // _R


def _sc_body(x_hbm, pos_hbm, out_hbm, pos_v, x_v):
    wid = lax.axis_index("s") * _NC + lax.axis_index("c")
    row0 = wid * _ROWS_W

    def chunk_body(c):
        seq_off = (row0 + c * _R) * _D
        pltpu.sync_copy(pos_hbm.at[pl.ds(seq_off, _CHUNK)], pos_v)

        def batch_body(b):
            flat_off = b * (_S * _D) + seq_off
            pltpu.sync_copy(x_hbm.at[pl.ds(flat_off, _CHUNK)], x_v)

            @pl.loop(0, _CHUNK // _L, unroll=8)
            def vec_body(i):
                sl = pl.ds(i * _L, _L)
                x_v[sl] = x_v[sl] + pos_v[sl]

            pltpu.sync_copy(x_v, out_hbm.at[pl.ds(flat_off, _CHUNK)])

        lax.fori_loop(0, _B, lambda b, _: (batch_body(b), 0)[1], 0)

    lax.fori_loop(0, _NCHUNK, lambda c, _: (chunk_body(c), 0)[1], 0)


def kernel(x, pos_emb):
    b, seq_len, dim = x.shape
    x_flat = x.reshape(b * seq_len * dim)
    pos_flat = pos_emb.reshape(seq_len * dim)
    mesh = plsc.VectorSubcoreMesh(core_axis_name="c", subcore_axis_name="s")
    out = pl.kernel(
        _sc_body,
        out_type=jax.ShapeDtypeStruct((b * seq_len * dim,), x.dtype),
        mesh=mesh,
        scratch_types=[
            pltpu.VMEM((_CHUNK,), jnp.float32),
            pltpu.VMEM((_CHUNK,), jnp.float32),
        ],
    )(x_flat, pos_flat)
    return out.reshape(b, seq_len, dim)


# speedup vs baseline: 1.1139x; 1.1139x over previous
import jax
import jax.numpy as jnp
from jax import lax
from jax.experimental import pallas as pl
from jax.experimental.pallas import tpu as pltpu
from jax.experimental.pallas import tpu_sc as plsc

_B, _S, _D = 4, 8192, 1024
_NC, _NS, _L = 2, 16, 16
_NW = _NC * _NS
_ROWS_W = _S // _NW
_R = 16
_CHUNK = _R * _D
_NCHUNK = _ROWS_W // _R
_T = _NCHUNK * _B
_NBUF = 4


def _sc_body(x_hbm, pos_hbm, out_hbm,
             p0, p1, x0, x1, x2, x3,
             psem, s0, s1, s2, s3, osem):
    wid = lax.axis_index("s") * _NC + lax.axis_index("c")
    row0 = wid * _ROWS_W
    xbuf = (x0, x1, x2, x3)
    isem = (s0, s1, s2, s3)
    pbuf = (p0, p1)

    def x_off(t):
        c, b = t // _B, t % _B
        return b * (_S * _D) + (row0 + c * _R) * _D

    def start_load(t):
        return pltpu.async_copy(
            x_hbm.at[pl.ds(x_off(t), _CHUNK)], xbuf[t % _NBUF], isem[t % _NBUF])

    def start_pos(c):
        return pltpu.async_copy(
            pos_hbm.at[pl.ds((row0 + c * _R) * _D, _CHUNK)], pbuf[c % 2], psem)

    in_d, out_d, pos_d = {}, {}, {}
    pos_d[0] = start_pos(0)
    for t in range(min(_NBUF - 1, _T)):
        in_d[t] = start_load(t)

    for t in range(_T):
        c, b = t // _B, t % _B
        if t >= 1:
            out_d[t - 1].wait()
        if t + _NBUF - 1 < _T:
            in_d[t + _NBUF - 1] = start_load(t + _NBUF - 1)
        if b == 0:
            pos_d[c].wait()
            if c + 1 < _NCHUNK:
                pos_d[c + 1] = start_pos(c + 1)
        in_d[t].wait()
        xv = xbuf[t % _NBUF]
        pv = pbuf[c % 2]

        @pl.loop(0, _CHUNK // _L, unroll=8)
        def vec_body(i):
            sl = pl.ds(i * _L, _L)
            xv[sl] = xv[sl] + pv[sl]

        out_d[t] = pltpu.async_copy(
            xv, out_hbm.at[pl.ds(x_off(t), _CHUNK)], osem)
    out_d[_T - 1].wait()


def kernel(x, pos_emb):
    b, seq_len, dim = x.shape
    x_flat = x.reshape(b * seq_len * dim)
    pos_flat = pos_emb.reshape(seq_len * dim)
    mesh = plsc.VectorSubcoreMesh(core_axis_name="c", subcore_axis_name="s")
    out = pl.kernel(
        _sc_body,
        out_type=jax.ShapeDtypeStruct((b * seq_len * dim,), x.dtype),
        mesh=mesh,
        scratch_types=[
            pltpu.VMEM((_CHUNK,), jnp.float32),
            pltpu.VMEM((_CHUNK,), jnp.float32),
            pltpu.VMEM((_CHUNK,), jnp.float32),
            pltpu.VMEM((_CHUNK,), jnp.float32),
            pltpu.VMEM((_CHUNK,), jnp.float32),
            pltpu.VMEM((_CHUNK,), jnp.float32),
            pltpu.SemaphoreType.DMA,
            pltpu.SemaphoreType.DMA,
            pltpu.SemaphoreType.DMA,
            pltpu.SemaphoreType.DMA,
            pltpu.SemaphoreType.DMA,
            pltpu.SemaphoreType.DMA,
        ],
    )(x_flat, pos_flat)
    return out.reshape(b, seq_len, dim)


# speedup vs baseline: 1.7427x; 1.5645x over previous
import jax
import jax.numpy as jnp
from jax import lax
from jax.experimental import pallas as pl
from jax.experimental.pallas import tpu as pltpu
from jax.experimental.pallas import tpu_sc as plsc

_B, _S, _D = 4, 8192, 1024
_NC, _NS, _L = 2, 16, 16
_NW = _NC * _NS
_ROWS_W = _S // _NW
_R = 16
_CHUNK = _R * _D
_NCHUNK = _ROWS_W // _R
_T = _NCHUNK * _B
_NBUF = 4


def _sc_body(x_hbm, pos_hbm, out_hbm,
             p0, p1, x0, x1, x2, x3,
             psem, s0, s1, s2, s3, osem):
    wid = lax.axis_index("s") * _NC + lax.axis_index("c")
    row0 = wid * _ROWS_W
    xbuf = (x0, x1, x2, x3)
    isem = (s0, s1, s2, s3)
    pbuf = (p0, p1)

    def x_off(t):
        c, b = t // _B, t % _B
        return b * (_S * _D) + (row0 + c * _R) * _D

    def start_load(t):
        return pltpu.async_copy(
            x_hbm.at[pl.ds(x_off(t), _CHUNK)], xbuf[t % _NBUF], isem[t % _NBUF])

    def start_pos(c):
        return pltpu.async_copy(
            pos_hbm.at[pl.ds((row0 + c * _R) * _D, _CHUNK)], pbuf[c % 2], psem)

    in_d, out_d, pos_d = {}, {}, {}
    pos_d[0] = start_pos(0)
    for t in range(min(_NBUF - 1, _T)):
        in_d[t] = start_load(t)

    for t in range(_T):
        c, b = t // _B, t % _B
        if t >= 1:
            out_d[t - 1].wait()
        if t + _NBUF - 1 < _T:
            in_d[t + _NBUF - 1] = start_load(t + _NBUF - 1)
        if b == 0:
            pos_d[c].wait()
            if c + 1 < _NCHUNK:
                pos_d[c + 1] = start_pos(c + 1)
        in_d[t].wait()
        xv = xbuf[t % _NBUF]
        pv = pbuf[c % 2]

        @plsc.parallel_loop(0, _CHUNK, step=_L, unroll=8)
        def vec_body(i):
            sl = pl.ds(i, _L)
            xv[sl] = xv[sl] + pv[sl]

        out_d[t] = pltpu.async_copy(
            xv, out_hbm.at[pl.ds(x_off(t), _CHUNK)], osem)
    out_d[_T - 1].wait()


def kernel(x, pos_emb):
    b, seq_len, dim = x.shape
    x_flat = x.reshape(b * seq_len * dim)
    pos_flat = pos_emb.reshape(seq_len * dim)
    mesh = plsc.VectorSubcoreMesh(core_axis_name="c", subcore_axis_name="s")
    out = pl.kernel(
        _sc_body,
        out_type=jax.ShapeDtypeStruct((b * seq_len * dim,), x.dtype),
        mesh=mesh,
        scratch_types=[
            pltpu.VMEM((_CHUNK,), jnp.float32),
            pltpu.VMEM((_CHUNK,), jnp.float32),
            pltpu.VMEM((_CHUNK,), jnp.float32),
            pltpu.VMEM((_CHUNK,), jnp.float32),
            pltpu.VMEM((_CHUNK,), jnp.float32),
            pltpu.VMEM((_CHUNK,), jnp.float32),
            pltpu.SemaphoreType.DMA,
            pltpu.SemaphoreType.DMA,
            pltpu.SemaphoreType.DMA,
            pltpu.SemaphoreType.DMA,
            pltpu.SemaphoreType.DMA,
            pltpu.SemaphoreType.DMA,
        ],
    )(x_flat, pos_flat)
    return out.reshape(b, seq_len, dim)


# speedup vs baseline: 7.5953x; 4.3582x over previous
import jax
import jax.numpy as jnp
from jax.experimental import pallas as pl
from jax.experimental.pallas import tpu as pltpu

_BS = 512


def _add_kernel(x_ref, pos_ref, out_ref):
    out_ref[...] = x_ref[...] + pos_ref[...]


def kernel(x, pos_emb):
    b, seq_len, dim = x.shape
    grid = (seq_len // _BS,)
    return pl.pallas_call(
        _add_kernel,
        grid=grid,
        in_specs=[
            pl.BlockSpec((b, _BS, dim), lambda s: (0, s, 0)),
            pl.BlockSpec((_BS, dim), lambda s: (s, 0)),
        ],
        out_specs=pl.BlockSpec((b, _BS, dim), lambda s: (0, s, 0)),
        out_shape=jax.ShapeDtypeStruct(x.shape, x.dtype),
        compiler_params=pltpu.CompilerParams(
            dimension_semantics=("parallel",),
        ),
    )(x, pos_emb)


# speedup vs baseline: 7.6161x; 1.0027x over previous
import jax
import jax.numpy as jnp
from jax.experimental import pallas as pl
from jax.experimental.pallas import tpu as pltpu

_BS = 1024
_BB = 2


def _add_kernel(x_ref, pos_ref, out_ref):
    out_ref[...] = x_ref[...] + pos_ref[...]


def kernel(x, pos_emb):
    b, seq_len, dim = x.shape
    grid = (seq_len // _BS, b // _BB)
    return pl.pallas_call(
        _add_kernel,
        grid=grid,
        in_specs=[
            pl.BlockSpec((_BB, _BS, dim), lambda s, bi: (bi, s, 0)),
            pl.BlockSpec((_BS, dim), lambda s, bi: (s, 0)),
        ],
        out_specs=pl.BlockSpec((_BB, _BS, dim), lambda s, bi: (bi, s, 0)),
        out_shape=jax.ShapeDtypeStruct(x.shape, x.dtype),
        compiler_params=pltpu.CompilerParams(
            dimension_semantics=("parallel", "arbitrary"),
        ),
    )(x, pos_emb)
